# Initial kernel scaffold; baseline (speedup 1.0000x reference)
import jax
import jax.numpy as jnp
from jax.experimental import pallas as pl


def kernel(row_ptr, edge_scores):
    raise NotImplementedError("write your pallas kernel here")


# speedup vs baseline: 2798.4716x; 2798.4716x over previous
import functools

import jax
import jax.numpy as jnp
from jax import lax
from jax.experimental import pallas as pl
from jax.experimental.pallas import tpu as pltpu
from jax.experimental.pallas import tpu_sc as plsc

N_NODES = 100000
N_EDGES = 6400000
K = 4

_INFO = plsc.get_sparse_core_info()
NC, NS, L = _INFO.num_cores, _INFO.num_subcores, _INFO.num_lanes
NW = NC * NS

NPW = -(-N_NODES // NW)
GPW = -(-NPW // L)
NPW_PAD = GPW * L
OUT_W = NPW_PAD * K

WIN = 2048
RP_WIN = ((NPW + 1 + 7) // 8 + 1) * 8

RP_PAD_LEN = ((NW - 1) * NPW // 8) * 8 + RP_WIN
E_PAD_LEN = N_EDGES + WIN

_NEG_INF = float("-inf")
_BIG = 0x3FFFFFFF


def _sc_topk(rp_hbm, ed_hbm, outv_hbm, outi_hbm, rp_v, win_v, ov_v, oi_v):
    wid = lax.axis_index("c") * NS + lax.axis_index("s")
    r0 = wid * NPW
    r0_al = pl.multiple_of(r0 & ~7, 8)
    off = r0 - r0_al
    pltpu.sync_copy(rp_hbm.at[pl.ds(r0_al, RP_WIN)], rp_v)

    lane = lax.iota(jnp.int32, L)

    def group_body(g, _):
        nl = g * L + lane
        valid = nl < NPW
        ridx = jnp.where(valid, off + nl, 0)
        s = plsc.load_gather(rp_v, [ridx])
        e = plsc.load_gather(rp_v, [ridx + 1])
        s = jnp.where(valid, s, 0)
        e = jnp.where(valid, e, 0)

        neg = jnp.full((L,), _NEG_INF, jnp.float32)
        mone = jnp.full((L,), -1, jnp.int32)
        carry = (s, neg, neg, neg, neg, mone, mone, mone, mone)

        def w_cond(c):
            pos = c[0]
            return jnp.max(e - pos) > 0

        def w_body(c):
            pos = c[0]
            pmin = jnp.min(jnp.where(pos < e, pos, _BIG))
            wlo = pl.multiple_of(pmin & ~7, 8)
            whi = wlo + WIN
            pltpu.sync_copy(ed_hbm.at[pl.ds(wlo, WIN)], win_v)
            steps = jnp.max(jnp.clip(jnp.minimum(e, whi) - pos, 0, None))

            def step(t, c2):
                pos, v0, v1, v2, v3, i0, i1, i2, i3 = c2
                active = (pos < e) & (pos < whi)
                rel = jnp.where(active, pos - wlo, 0)
                val = plsc.load_gather(win_v, [rel], mask=active)
                val = jnp.where(active, val, neg)
                b0 = val > v0
                b1 = val > v1
                b2 = val > v2
                b3 = val > v3
                nv0 = jnp.where(b0, val, v0)
                ni0 = jnp.where(b0, pos, i0)
                nv1 = jnp.where(b0, v0, jnp.where(b1, val, v1))
                ni1 = jnp.where(b0, i0, jnp.where(b1, pos, i1))
                nv2 = jnp.where(b1, v1, jnp.where(b2, val, v2))
                ni2 = jnp.where(b1, i1, jnp.where(b2, pos, i2))
                nv3 = jnp.where(b2, v2, jnp.where(b3, val, v3))
                ni3 = jnp.where(b2, i2, jnp.where(b3, pos, i3))
                pos = pos + active.astype(jnp.int32)
                return (pos, nv0, nv1, nv2, nv3, ni0, ni1, ni2, ni3)

            return lax.fori_loop(0, steps, step, c)

        res = lax.while_loop(w_cond, w_body, carry)
        _, v0, v1, v2, v3, i0, i1, i2, i3 = res

        addr = nl * K
        plsc.store_scatter(ov_v, [addr], v0)
        plsc.store_scatter(ov_v, [addr + 1], v1)
        plsc.store_scatter(ov_v, [addr + 2], v2)
        plsc.store_scatter(ov_v, [addr + 3], v3)
        plsc.store_scatter(oi_v, [addr], i0)
        plsc.store_scatter(oi_v, [addr + 1], i1)
        plsc.store_scatter(oi_v, [addr + 2], i2)
        plsc.store_scatter(oi_v, [addr + 3], i3)
        return 0

    lax.fori_loop(0, GPW, group_body, 0)

    pltpu.sync_copy(ov_v, outv_hbm.at[wid])
    pltpu.sync_copy(oi_v, outi_hbm.at[wid])


_sc_call = functools.partial(
    pl.kernel,
    mesh=plsc.VectorSubcoreMesh(core_axis_name="c", subcore_axis_name="s"),
    compiler_params=pltpu.CompilerParams(needs_layout_passes=False),
    out_type=[
        jax.ShapeDtypeStruct((NW, OUT_W), jnp.float32),
        jax.ShapeDtypeStruct((NW, OUT_W), jnp.int32),
    ],
    scratch_types=[
        pltpu.VMEM((RP_WIN,), jnp.int32),
        pltpu.VMEM((WIN,), jnp.float32),
        pltpu.VMEM((OUT_W,), jnp.float32),
        pltpu.VMEM((OUT_W,), jnp.int32),
    ],
)(_sc_topk)


@jax.jit
def kernel(row_ptr, edge_scores):
    rp = jnp.concatenate(
        [row_ptr.astype(jnp.int32),
         jnp.zeros((RP_PAD_LEN - (N_NODES + 1),), jnp.int32)]
    )
    ed = jnp.concatenate(
        [edge_scores, jnp.zeros((E_PAD_LEN - N_EDGES,), jnp.float32)]
    )
    outv, outi = _sc_call(rp, ed)
    vals = outv.reshape(NW, NPW_PAD, K)[:, :NPW, :].reshape(N_NODES, K)
    idxs = outi.reshape(NW, NPW_PAD, K)[:, :NPW, :].reshape(N_NODES, K)
    return vals, idxs.astype(jnp.int64)


# speedup vs baseline: 4424.9505x; 1.5812x over previous
import functools

import jax
import jax.numpy as jnp
from jax import lax
from jax.experimental import pallas as pl
from jax.experimental.pallas import tpu as pltpu
from jax.experimental.pallas import tpu_sc as plsc

N_NODES = 100000
N_EDGES = 6400000
K = 4

_INFO = plsc.get_sparse_core_info()
NC, NS, L = _INFO.num_cores, _INFO.num_subcores, _INFO.num_lanes
NW = NC * NS

NPW = -(-N_NODES // NW)
CHW = -(-NPW // L)
NPW_PAD = CHW * L
OUT_W = NPW_PAD * K

ROW_LOG2 = 10
ROW_W = 1 << ROW_LOG2
EP_ROWS = N_EDGES // ROW_W + 2
E_PAD_LEN = EP_ROWS * ROW_W

PHASE_BLOCKS = ROW_W // 4

RP_WIN = ((NPW_PAD + 1 + 7) // 8 + 1) * 8
RP_PAD_LEN = ((NW - 1) * NPW // 8) * 8 + RP_WIN

_NEG_INF = float("-inf")


def _sc_topk(rp_hbm, ed_hbm, outv_hbm, outi_hbm, rp_v, buf_v, idx_v, ov_v, oi_v, sem):
    wid = lax.axis_index("c") * NS + lax.axis_index("s")
    r0 = wid * NPW
    r0_al = pl.multiple_of(r0 & ~7, 8)
    off = r0 - r0_al
    pltpu.sync_copy(rp_hbm.at[pl.ds(r0_al, RP_WIN)], rp_v)

    lane = lax.iota(jnp.int32, L)
    neg = jnp.full((L,), _NEG_INF, jnp.float32)
    mone = jnp.full((L,), -1, jnp.int32)

    nl0 = lane * CHW
    lend = nl0 + CHW
    pos0 = plsc.load_gather(rp_v, [off + nl0])
    e0 = plsc.load_gather(rp_v, [off + nl0 + 1])

    def step(pos, e, base, brow, v0, v1, v2, v3, i0, i1, i2, i3):
        active = pos < e
        rel2 = pos - base
        row = lane + ((rel2 >> ROW_LOG2) << 4)
        col = rel2 & (ROW_W - 1)
        val = plsc.load_gather(buf_v, [row, col], mask=active)
        val = jnp.where(active, val, neg)
        b0 = val > v0
        b1 = val > v1
        b2 = val > v2
        b3 = val > v3
        nv0 = jnp.where(b0, val, v0)
        ni0 = jnp.where(b0, pos, i0)
        nv1 = jnp.where(b0, v0, jnp.where(b1, val, v1))
        ni1 = jnp.where(b0, i0, jnp.where(b1, pos, i1))
        nv2 = jnp.where(b1, v1, jnp.where(b2, val, v2))
        ni2 = jnp.where(b1, i1, jnp.where(b2, pos, i2))
        nv3 = jnp.where(b2, v2, jnp.where(b3, val, v3))
        ni3 = jnp.where(b2, i2, jnp.where(b3, pos, i3))
        pos = pos + active.astype(jnp.int32)
        return pos, nv0, nv1, nv2, nv3, ni0, ni1, ni2, ni3

    def phase_cond(c):
        nl = c[0]
        return jnp.max(lend - nl) > 0

    def phase_body(c):
        nl, pos, e, v0, v1, v2, v3, i0, i1, i2, i3 = c
        brow = pos >> ROW_LOG2
        idx_v[pl.ds(0, L)] = brow
        idx_v[pl.ds(L, L)] = brow + 1
        pltpu.async_copy(ed_hbm.at[idx_v], buf_v, sem).wait()
        base = brow << ROW_LOG2

        def block(b, c2):
            nl, pos, e, v0, v1, v2, v3, i0, i1, i2, i3 = c2
            done = (pos >= e) & (nl < lend)
            addr = nl * K
            plsc.store_scatter(ov_v, [addr], v0, mask=done)
            plsc.store_scatter(ov_v, [addr + 1], v1, mask=done)
            plsc.store_scatter(ov_v, [addr + 2], v2, mask=done)
            plsc.store_scatter(ov_v, [addr + 3], v3, mask=done)
            plsc.store_scatter(oi_v, [addr], i0, mask=done)
            plsc.store_scatter(oi_v, [addr + 1], i1, mask=done)
            plsc.store_scatter(oi_v, [addr + 2], i2, mask=done)
            plsc.store_scatter(oi_v, [addr + 3], i3, mask=done)
            nl = nl + done.astype(jnp.int32)
            eg = plsc.load_gather(rp_v, [off + nl + 1])
            at_end = nl >= lend
            e = jnp.where(done, jnp.where(at_end, pos, eg), e)
            v0 = jnp.where(done, neg, v0)
            v1 = jnp.where(done, neg, v1)
            v2 = jnp.where(done, neg, v2)
            v3 = jnp.where(done, neg, v3)
            i0 = jnp.where(done, mone, i0)
            i1 = jnp.where(done, mone, i1)
            i2 = jnp.where(done, mone, i2)
            i3 = jnp.where(done, mone, i3)
            st = (pos, v0, v1, v2, v3, i0, i1, i2, i3)
            st = step(st[0], e, base, brow, *st[1:])
            st = step(st[0], e, base, brow, *st[1:])
            st = step(st[0], e, base, brow, *st[1:])
            st = step(st[0], e, base, brow, *st[1:])
            pos, v0, v1, v2, v3, i0, i1, i2, i3 = st
            return (nl, pos, e, v0, v1, v2, v3, i0, i1, i2, i3)

        return lax.fori_loop(0, PHASE_BLOCKS, block, c)

    carry = (nl0, pos0, e0, neg, neg, neg, neg, mone, mone, mone, mone)
    lax.while_loop(phase_cond, phase_body, carry)

    pltpu.sync_copy(ov_v, outv_hbm.at[wid])
    pltpu.sync_copy(oi_v, outi_hbm.at[wid])


_sc_call = functools.partial(
    pl.kernel,
    mesh=plsc.VectorSubcoreMesh(core_axis_name="c", subcore_axis_name="s"),
    compiler_params=pltpu.CompilerParams(needs_layout_passes=False),
    out_type=[
        jax.ShapeDtypeStruct((NW, OUT_W), jnp.float32),
        jax.ShapeDtypeStruct((NW, OUT_W), jnp.int32),
    ],
    scratch_types=[
        pltpu.VMEM((RP_WIN,), jnp.int32),
        pltpu.VMEM((2 * L, ROW_W), jnp.float32),
        pltpu.VMEM((2 * L,), jnp.int32),
        pltpu.VMEM((OUT_W,), jnp.float32),
        pltpu.VMEM((OUT_W,), jnp.int32),
        pltpu.SemaphoreType.DMA,
    ],
)(_sc_topk)


@jax.jit
def kernel(row_ptr, edge_scores):
    rp = jnp.concatenate(
        [row_ptr.astype(jnp.int32),
         jnp.zeros((RP_PAD_LEN - (N_NODES + 1),), jnp.int32)]
    )
    ed = jnp.concatenate(
        [edge_scores, jnp.zeros((E_PAD_LEN - N_EDGES,), jnp.float32)]
    ).reshape(EP_ROWS, ROW_W)
    outv, outi = _sc_call(rp, ed)
    vals = outv.reshape(NW, NPW_PAD, K)[:, :NPW, :].reshape(N_NODES, K)
    idxs = outi.reshape(NW, NPW_PAD, K)[:, :NPW, :].reshape(N_NODES, K)
    return vals, idxs.astype(jnp.int64)


# speedup vs baseline: 4518.8156x; 1.0212x over previous
import functools

import jax
import jax.numpy as jnp
from jax import lax
from jax.experimental import pallas as pl
from jax.experimental.pallas import tpu as pltpu
from jax.experimental.pallas import tpu_sc as plsc

N_NODES = 100000
N_EDGES = 6400000
K = 4

_INFO = plsc.get_sparse_core_info()
NC, NS, L = _INFO.num_cores, _INFO.num_subcores, _INFO.num_lanes
NW = NC * NS

NPW = -(-N_NODES // NW)
CHW = -(-NPW // L)
NPW_PAD = CHW * L
OUT_W = NPW_PAD * K

ROW_LOG2 = 11
ROW_W = 1 << ROW_LOG2
EP_ROWS = N_EDGES // ROW_W + 2
E_PAD_LEN = EP_ROWS * ROW_W

BLK = 8
PHASE_BLOCKS = ROW_W // BLK

RP_WIN = ((NPW_PAD + 1 + 7) // 8 + 1) * 8
RP_PAD_LEN = ((NW - 1) * NPW // 8) * 8 + RP_WIN

_NEG_INF = float("-inf")


def _sc_topk(rp_hbm, ed_hbm, outv_hbm, outi_hbm, rp_v, buf_v, idx_v, ov_v, oi_v, sem):
    wid = lax.axis_index("c") * NS + lax.axis_index("s")
    r0 = wid * NPW
    r0_al = pl.multiple_of(r0 & ~7, 8)
    off = r0 - r0_al
    pltpu.sync_copy(rp_hbm.at[pl.ds(r0_al, RP_WIN)], rp_v)

    lane = lax.iota(jnp.int32, L)
    lane2 = lane * 2
    neg = jnp.full((L,), _NEG_INF, jnp.float32)
    mone = jnp.full((L,), -1, jnp.int32)

    nl0 = lane * CHW
    lend = nl0 + CHW
    pos0 = plsc.load_gather(rp_v, [off + nl0])
    e0 = plsc.load_gather(rp_v, [off + nl0 + 1])

    def step(e, base, pos, v0, v1, v2, v3, i0, i1, i2, i3):
        active = pos < e
        rel2 = pos - base
        row = lane2 + (rel2 >> ROW_LOG2)
        col = rel2 & (ROW_W - 1)
        val = plsc.load_gather(buf_v, [row, col], mask=active)
        val = jnp.where(active, val, neg)
        b0 = val > v0
        b1 = val > v1
        b2 = val > v2
        b3 = val > v3
        nv0 = jnp.where(b0, val, v0)
        ni0 = jnp.where(b0, pos, i0)
        nv1 = jnp.where(b0, v0, jnp.where(b1, val, v1))
        ni1 = jnp.where(b0, i0, jnp.where(b1, pos, i1))
        nv2 = jnp.where(b1, v1, jnp.where(b2, val, v2))
        ni2 = jnp.where(b1, i1, jnp.where(b2, pos, i2))
        nv3 = jnp.where(b2, v2, jnp.where(b3, val, v3))
        ni3 = jnp.where(b2, i2, jnp.where(b3, pos, i3))
        pos = pos + active.astype(jnp.int32)
        return pos, nv0, nv1, nv2, nv3, ni0, ni1, ni2, ni3

    def phase_cond(c):
        nl = c[0]
        return jnp.max(lend - nl) > 0

    def phase_body(c):
        nl, pos, e, v0, v1, v2, v3, i0, i1, i2, i3 = c
        brow = pos >> ROW_LOG2
        plsc.store_scatter(idx_v, [lane2], brow)
        plsc.store_scatter(idx_v, [lane2 + 1], brow + 1)
        pltpu.async_copy(ed_hbm.at[idx_v], buf_v, sem).wait()
        base = brow << ROW_LOG2

        def block(b, c2):
            nl, pos, e, v0, v1, v2, v3, i0, i1, i2, i3 = c2
            done = (pos >= e) & (nl < lend)
            addr = nl * K
            plsc.store_scatter(ov_v, [addr], v0, mask=done)
            plsc.store_scatter(ov_v, [addr + 1], v1, mask=done)
            plsc.store_scatter(ov_v, [addr + 2], v2, mask=done)
            plsc.store_scatter(ov_v, [addr + 3], v3, mask=done)
            plsc.store_scatter(oi_v, [addr], i0, mask=done)
            plsc.store_scatter(oi_v, [addr + 1], i1, mask=done)
            plsc.store_scatter(oi_v, [addr + 2], i2, mask=done)
            plsc.store_scatter(oi_v, [addr + 3], i3, mask=done)
            nl = nl + done.astype(jnp.int32)
            eg = plsc.load_gather(rp_v, [off + nl + 1])
            at_end = nl >= lend
            e = jnp.where(done, jnp.where(at_end, pos, eg), e)
            v0 = jnp.where(done, neg, v0)
            v1 = jnp.where(done, neg, v1)
            v2 = jnp.where(done, neg, v2)
            v3 = jnp.where(done, neg, v3)
            i0 = jnp.where(done, mone, i0)
            i1 = jnp.where(done, mone, i1)
            i2 = jnp.where(done, mone, i2)
            i3 = jnp.where(done, mone, i3)
            st = (pos, v0, v1, v2, v3, i0, i1, i2, i3)
            for _ in range(BLK):
                st = step(e, base, *st)
            pos, v0, v1, v2, v3, i0, i1, i2, i3 = st
            return (nl, pos, e, v0, v1, v2, v3, i0, i1, i2, i3)

        return lax.fori_loop(0, PHASE_BLOCKS, block, c)

    carry = (nl0, pos0, e0, neg, neg, neg, neg, mone, mone, mone, mone)
    lax.while_loop(phase_cond, phase_body, carry)

    pltpu.sync_copy(ov_v, outv_hbm.at[wid])
    pltpu.sync_copy(oi_v, outi_hbm.at[wid])


_sc_call = functools.partial(
    pl.kernel,
    mesh=plsc.VectorSubcoreMesh(core_axis_name="c", subcore_axis_name="s"),
    compiler_params=pltpu.CompilerParams(needs_layout_passes=False),
    out_type=[
        jax.ShapeDtypeStruct((NW, OUT_W), jnp.float32),
        jax.ShapeDtypeStruct((NW, OUT_W), jnp.int32),
    ],
    scratch_types=[
        pltpu.VMEM((RP_WIN,), jnp.int32),
        pltpu.VMEM((2 * L, ROW_W), jnp.float32),
        pltpu.VMEM((2 * L,), jnp.int32),
        pltpu.VMEM((OUT_W,), jnp.float32),
        pltpu.VMEM((OUT_W,), jnp.int32),
        pltpu.SemaphoreType.DMA,
    ],
)(_sc_topk)


@jax.jit
def kernel(row_ptr, edge_scores):
    rp = jnp.concatenate(
        [row_ptr.astype(jnp.int32),
         jnp.zeros((RP_PAD_LEN - (N_NODES + 1),), jnp.int32)]
    )
    ed = jnp.concatenate(
        [edge_scores, jnp.zeros((E_PAD_LEN - N_EDGES,), jnp.float32)]
    ).reshape(EP_ROWS, ROW_W)
    outv, outi = _sc_call(rp, ed)
    vals = outv.reshape(NW, NPW_PAD, K)[:, :NPW, :].reshape(N_NODES, K)
    idxs = outi.reshape(NW, NPW_PAD, K)[:, :NPW, :].reshape(N_NODES, K)
    return vals, idxs.astype(jnp.int64)
